# baseline (speedup 1.0000x reference)
import functools

import jax
import jax.numpy as jnp
from jax import lax
from jax.experimental import pallas as pl
from jax.experimental.pallas import tpu as pltpu
from jax.experimental.pallas import tpu_sc as plsc

_LANES = 16


def _broadcast_add_sc(x_flat, mid, embedding):
    n_words = x_flat.shape[0]
    _, d = embedding.shape
    vecs_per_row = d // _LANES

    info = plsc.get_sparse_core_info()
    nc, ns = info.num_cores, info.num_subcores
    nw = nc * ns
    words_per_w = n_words // nw
    chunk_words = 32 * d
    n_chunks = words_per_w // chunk_words
    nbuf = 3

    mesh = plsc.VectorSubcoreMesh(core_axis_name="c", subcore_axis_name="s")

    @functools.partial(
        pl.kernel,
        mesh=mesh,
        out_type=jax.ShapeDtypeStruct((n_words,), jnp.float32),
        scratch_types=[
            pltpu.VMEM((1,), jnp.int32),
            pltpu.VMEM((1, d), jnp.float32),
            [pltpu.VMEM((chunk_words,), jnp.float32) for _ in range(nbuf)],
            [pltpu.SemaphoreType.DMA for _ in range(nbuf)],
            [pltpu.SemaphoreType.DMA for _ in range(nbuf)],
            pltpu.SemaphoreType.DMA,
        ],
    )
    def run(x_hbm, mid_hbm, emb_hbm, out_hbm, idx_v, emb_v, bufs, isems,
            osems, gsem):
        wid = lax.axis_index("s") * nc + lax.axis_index("c")
        base = wid * words_per_w

        pltpu.sync_copy(mid_hbm, idx_v)
        pltpu.async_copy(emb_hbm.at[idx_v], emb_v, gsem).wait()

        def start_in(i):
            off = base + i * chunk_words
            return pltpu.async_copy(
                x_hbm.at[pl.ds(off, chunk_words)], bufs[i % nbuf],
                isems[i % nbuf])

        def start_out(i):
            off = base + i * chunk_words
            return pltpu.async_copy(
                bufs[i % nbuf], out_hbm.at[pl.ds(off, chunk_words)],
                osems[i % nbuf])

        in_dma = {}
        out_dma = {}
        for i in range(min(2, n_chunks)):
            in_dma[i] = start_in(i)

        for i in range(n_chunks):
            buf = bufs[i % nbuf]
            in_dma.pop(i).wait()

            def add_body(j, carry):
                k16 = (j % vecs_per_row) * _LANES
                sl = pl.ds(j * _LANES, _LANES)
                buf[sl] = buf[sl] + emb_v[0, pl.ds(k16, _LANES)]
                return carry

            lax.fori_loop(0, chunk_words // _LANES, add_body, 0)

            out_dma[i] = start_out(i)
            if i + 2 < n_chunks:
                if i - 1 >= 0:
                    out_dma.pop(i - 1).wait()
                in_dma[i + 2] = start_in(i + 2)

        for i in sorted(out_dma):
            out_dma[i].wait()

    return run


def kernel(x, modality_id, embedding):
    b, t, d = x.shape
    x_flat = x.reshape(b * t * d)
    mid = jnp.asarray(modality_id, jnp.int32).reshape(1)
    out_flat = _broadcast_add_sc(x_flat, mid, embedding)(
        x_flat, mid, embedding)
    return out_flat.reshape(b, t, d)


# speedup vs baseline: 1.6895x; 1.6895x over previous
import functools

import jax
import jax.numpy as jnp
from jax import lax
from jax.experimental import pallas as pl
from jax.experimental.pallas import tpu as pltpu
from jax.experimental.pallas import tpu_sc as plsc

_LANES = 16


def _broadcast_add_sc(x_flat, mid, embedding):
    n_words = x_flat.shape[0]
    _, d = embedding.shape
    vecs_per_row = d // _LANES

    info = plsc.get_sparse_core_info()
    nc, ns = info.num_cores, info.num_subcores
    nw = nc * ns
    words_per_w = n_words // nw
    chunk_words = 32 * d
    n_chunks = words_per_w // chunk_words
    nbuf = 3

    mesh = plsc.VectorSubcoreMesh(core_axis_name="c", subcore_axis_name="s")

    @functools.partial(
        pl.kernel,
        mesh=mesh,
        out_type=jax.ShapeDtypeStruct((n_words,), jnp.float32),
        scratch_types=[
            pltpu.VMEM((1,), jnp.int32),
            pltpu.VMEM((1, d), jnp.float32),
            [pltpu.VMEM((chunk_words,), jnp.float32) for _ in range(nbuf)],
            [pltpu.SemaphoreType.DMA for _ in range(nbuf)],
            [pltpu.SemaphoreType.DMA for _ in range(nbuf)],
            pltpu.SemaphoreType.DMA,
        ],
    )
    def run(x_hbm, mid_hbm, emb_hbm, out_hbm, idx_v, emb_v, bufs, isems,
            osems, gsem):
        wid = lax.axis_index("s") * nc + lax.axis_index("c")
        base = wid * words_per_w

        pltpu.sync_copy(mid_hbm, idx_v)
        pltpu.async_copy(emb_hbm.at[idx_v], emb_v, gsem).wait()

        def start_in(i):
            off = base + i * chunk_words
            return pltpu.async_copy(
                x_hbm.at[pl.ds(off, chunk_words)], bufs[i % nbuf],
                isems[i % nbuf])

        def start_out(i):
            off = base + i * chunk_words
            return pltpu.async_copy(
                bufs[i % nbuf], out_hbm.at[pl.ds(off, chunk_words)],
                osems[i % nbuf])

        in_dma = {}
        out_dma = {}
        for i in range(min(2, n_chunks)):
            in_dma[i] = start_in(i)

        chunk_rows = chunk_words // d
        half = vecs_per_row // 2

        for i in range(n_chunks):
            buf = bufs[i % nbuf]
            in_dma.pop(i).wait()

            for h in range(2):
                evs = [
                    emb_v[0, pl.ds((h * half + k) * _LANES, _LANES)]
                    for k in range(half)
                ]

                def row_body(r, carry, evs=evs, buf=buf, h=h):
                    base = r * d + h * half * _LANES
                    for k in range(half):
                        sl = pl.ds(base + k * _LANES, _LANES)
                        buf[sl] = buf[sl] + evs[k]
                    return carry

                lax.fori_loop(0, chunk_rows, row_body, 0)

            out_dma[i] = start_out(i)
            if i + 2 < n_chunks:
                if i - 1 >= 0:
                    out_dma.pop(i - 1).wait()
                in_dma[i + 2] = start_in(i + 2)

        for i in sorted(out_dma):
            out_dma[i].wait()

    return run


def kernel(x, modality_id, embedding):
    b, t, d = x.shape
    x_flat = x.reshape(b * t * d)
    mid = jnp.asarray(modality_id, jnp.int32).reshape(1)
    out_flat = _broadcast_add_sc(x_flat, mid, embedding)(
        x_flat, mid, embedding)
    return out_flat.reshape(b, t, d)


# speedup vs baseline: 4.2150x; 2.4948x over previous
import functools

import jax
import jax.numpy as jnp
from jax import lax
from jax.experimental import pallas as pl
from jax.experimental.pallas import tpu as pltpu
from jax.experimental.pallas import tpu_sc as plsc

_LANES = 16


def _broadcast_add_sc(x2, mid, embedding):
    n_rows, d = x2.shape
    vecs_per_row = d // _LANES

    info = plsc.get_sparse_core_info()
    nc, ns = info.num_cores, info.num_subcores
    nw = nc * ns
    rows_per_w = n_rows // nw
    chunk_rows = 32
    n_chunks = rows_per_w // chunk_rows
    nbuf = 3

    mesh = plsc.VectorSubcoreMesh(core_axis_name="c", subcore_axis_name="s")

    @functools.partial(
        pl.kernel,
        mesh=mesh,
        out_type=jax.ShapeDtypeStruct((n_rows, d), jnp.float32),
        scratch_types=[
            pltpu.VMEM((1,), jnp.int32),
            pltpu.VMEM((1, d), jnp.float32),
            [pltpu.VMEM((chunk_rows, d), jnp.float32) for _ in range(nbuf)],
            [pltpu.SemaphoreType.DMA for _ in range(nbuf)],
            [pltpu.SemaphoreType.DMA for _ in range(nbuf)],
            pltpu.SemaphoreType.DMA,
        ],
    )
    def run(x_hbm, mid_hbm, emb_hbm, out_hbm, idx_v, emb_v, bufs, isems,
            osems, gsem):
        wid = lax.axis_index("s") * nc + lax.axis_index("c")
        base = wid * rows_per_w

        pltpu.sync_copy(mid_hbm, idx_v)
        pltpu.async_copy(emb_hbm.at[idx_v], emb_v, gsem).wait()

        def start_in(i):
            off = base + i * chunk_rows
            return pltpu.async_copy(
                x_hbm.at[pl.ds(off, chunk_rows)], bufs[i % nbuf],
                isems[i % nbuf])

        def start_out(i):
            off = base + i * chunk_rows
            return pltpu.async_copy(
                bufs[i % nbuf], out_hbm.at[pl.ds(off, chunk_rows)],
                osems[i % nbuf])

        half = vecs_per_row // 2

        in_dma = {}
        out_dma = {}
        for i in range(min(2, n_chunks)):
            in_dma[i] = start_in(i)

        for i in range(n_chunks):
            buf = bufs[i % nbuf]
            in_dma.pop(i).wait()

            for h in range(2):
                evs = [
                    emb_v[0, pl.ds((h * half + k) * _LANES, _LANES)]
                    for k in range(half)
                ]

                def row_body(r, carry, evs=evs, buf=buf, h=h):
                    for k in range(half):
                        sl = pl.ds((h * half + k) * _LANES, _LANES)
                        buf[r, sl] = buf[r, sl] + evs[k]
                    return carry

                lax.fori_loop(0, chunk_rows, row_body, 0)

            out_dma[i] = start_out(i)
            if i + 2 < n_chunks:
                if i - 1 >= 0:
                    out_dma.pop(i - 1).wait()
                in_dma[i + 2] = start_in(i + 2)

        for i in sorted(out_dma):
            out_dma[i].wait()

    return run


def kernel(x, modality_id, embedding):
    b, t, d = x.shape
    x2 = x.reshape(b * t, d)
    mid = jnp.asarray(modality_id, jnp.int32).reshape(1)
    out2 = _broadcast_add_sc(x2, mid, embedding)(x2, mid, embedding)
    return out2.reshape(b, t, d)


# speedup vs baseline: 4.4512x; 1.0560x over previous
import functools

import jax
import jax.numpy as jnp
from jax import lax
from jax.experimental import pallas as pl
from jax.experimental.pallas import tpu as pltpu
from jax.experimental.pallas import tpu_sc as plsc

_LANES = 16


def _broadcast_add_sc_rows(x2, mid, embedding, n_sc):
    _, d = x2.shape
    vecs_per_row = d // _LANES

    info = plsc.get_sparse_core_info()
    nc, ns = info.num_cores, info.num_subcores
    nw = nc * ns
    rows_per_w = n_sc // nw
    chunk_rows = 32
    half_rows = chunk_rows // 2
    n_chunks = rows_per_w // chunk_rows
    nbuf = 3

    mesh = plsc.VectorSubcoreMesh(core_axis_name="c", subcore_axis_name="s")

    @functools.partial(
        pl.kernel,
        mesh=mesh,
        out_type=jax.ShapeDtypeStruct((n_sc, d), jnp.float32),
        scratch_types=[
            pltpu.VMEM((1,), jnp.int32),
            pltpu.VMEM((1, d), jnp.float32),
            [pltpu.VMEM((chunk_rows, d), jnp.float32) for _ in range(nbuf)],
            [pltpu.SemaphoreType.DMA for _ in range(nbuf)],
            [pltpu.SemaphoreType.DMA for _ in range(nbuf)],
            [pltpu.SemaphoreType.DMA for _ in range(nbuf)],
            pltpu.SemaphoreType.DMA,
        ],
    )
    def run(x_hbm, mid_hbm, emb_hbm, out_hbm, idx_v, emb_v, bufs, isems,
            isems2, osems, gsem):
        wid = lax.axis_index("s") * nc + lax.axis_index("c")
        base = wid * rows_per_w

        pltpu.sync_copy(mid_hbm, idx_v)
        pltpu.async_copy(emb_hbm.at[idx_v], emb_v, gsem).wait()

        def start_in(i):
            off = base + i * chunk_rows
            b = i % nbuf
            d1 = pltpu.async_copy(
                x_hbm.at[pl.ds(off, half_rows)],
                bufs[b].at[pl.ds(0, half_rows)], isems[b])
            d2 = pltpu.async_copy(
                x_hbm.at[pl.ds(off + half_rows, half_rows)],
                bufs[b].at[pl.ds(half_rows, half_rows)], isems2[b])
            return (d1, d2)

        def start_out(i):
            off = base + i * chunk_rows
            return pltpu.async_copy(
                bufs[i % nbuf], out_hbm.at[pl.ds(off, chunk_rows)],
                osems[i % nbuf])

        half = vecs_per_row // 2

        depth = nbuf - 1
        in_dma = {}
        out_dma = {}
        for i in range(min(depth, n_chunks)):
            in_dma[i] = start_in(i)

        for i in range(n_chunks):
            buf = bufs[i % nbuf]
            d1, d2 = in_dma.pop(i)

            for rh, dma in ((0, d1), (1, d2)):
                dma.wait()
                r0 = rh * half_rows

                def col_body(k, carry, buf=buf, r0=r0):
                    sl = pl.ds(k * _LANES, _LANES)
                    ev = emb_v[0, sl]
                    for r in range(half_rows):
                        buf[r0 + r, sl] = buf[r0 + r, sl] + ev
                    return carry

                lax.fori_loop(0, vecs_per_row, col_body, 0)

            out_dma[i] = start_out(i)
            if i + depth < n_chunks:
                if i - 1 >= 0:
                    out_dma.pop(i - 1).wait()
                in_dma[i + depth] = start_in(i + depth)

        for i in sorted(out_dma):
            out_dma[i].wait()

    return run


def kernel(x, modality_id, embedding):
    b, t, d = x.shape
    x2 = x.reshape(b * t, d)
    mid = jnp.asarray(modality_id, jnp.int32).reshape(1)
    out2 = _broadcast_add_sc_rows(x2, mid, embedding, b * t)(
        x2, mid, embedding)
    return out2.reshape(b, t, d)
